# baseline (speedup 1.0000x reference)
import functools

import jax
import jax.numpy as jnp
from jax import lax
from jax.experimental import pallas as pl
from jax.experimental.pallas import tpu as pltpu
from jax.experimental.pallas import tpu_sc as plsc

N_VOCAB = 100000
N_CTX = 2048
N_EMBED = 128
BATCH = 4

NC = 2
NS = 16
NW = NC * NS
NTOK = BATCH * N_CTX
BPW = NTOK // NW
WPB = N_CTX // BPW
GCH = 32
NG = BPW // GCH


def _sc_embed(x_hbm, wte_hbm, wpe_hbm, out_hbm, idx_v, rows_v, sem_i, sem_p, sem_g, sem_s):
    wid = lax.axis_index("s") * NC + lax.axis_index("c")
    b = wid // WPB
    t0 = lax.rem(wid, WPB) * BPW

    cp_i = [
        pltpu.async_copy(x_hbm.at[b, pl.ds(t0 + g * GCH, GCH)], idx_v.at[g], sem_i)
        for g in range(NG)
    ]
    cp_p = [
        pltpu.async_copy(
            wpe_hbm.at[pl.ds(t0 + g * GCH, GCH)],
            rows_v.at[pl.ds(g * GCH, GCH)],
            sem_p,
        )
        for g in range(NG)
    ]
    cp_g = []
    for g in range(NG):
        cp_i[g].wait()
        cp_p[g].wait()
        cp_g.append(
            pltpu.async_copy(
                wte_hbm.at[idx_v.at[g]],
                rows_v.at[pl.ds(g * GCH, GCH)],
                sem_g,
                add=True,
            )
        )
    cp_s = []
    for g in range(NG):
        cp_g[g].wait()
        cp_s.append(
            pltpu.async_copy(
                rows_v.at[pl.ds(g * GCH, GCH)],
                out_hbm.at[b, pl.ds(t0 + g * GCH, GCH)],
                sem_s,
            )
        )
    for cp in cp_s:
        cp.wait()


@jax.jit
def _embed(x, wte, wpe):
    mesh = plsc.VectorSubcoreMesh(core_axis_name="c", subcore_axis_name="s")
    run = functools.partial(
        pl.kernel,
        out_type=jax.ShapeDtypeStruct((BATCH, N_CTX, N_EMBED), jnp.float32),
        mesh=mesh,
        scratch_types=[
            pltpu.VMEM((NG, GCH), jnp.int32),
            pltpu.VMEM((BPW, N_EMBED), jnp.float32),
            pltpu.SemaphoreType.DMA,
            pltpu.SemaphoreType.DMA,
            pltpu.SemaphoreType.DMA,
            pltpu.SemaphoreType.DMA,
        ],
    )(_sc_embed)
    return run(x, wte, wpe)


def kernel(x, wte, wpe):
    return _embed(x.astype(jnp.int32), wte, wpe)


# speedup vs baseline: 1.0369x; 1.0369x over previous
import functools

import jax
import jax.numpy as jnp
from jax import lax
from jax.experimental import pallas as pl
from jax.experimental.pallas import tpu as pltpu
from jax.experimental.pallas import tpu_sc as plsc

N_VOCAB = 100000
N_CTX = 2048
N_EMBED = 128
BATCH = 4

NC = 2
NS = 16
NW = NC * NS
PPW = N_CTX // NW


def _sc_embed(x_hbm, wte_hbm, wpe_hbm, out_hbm, idx_v, wv, rows_v, sem_i, sem_p, sem_c, sem_g, sem_s):
    wid = lax.axis_index("s") * NC + lax.axis_index("c")
    sid = lax.axis_index("s")
    p0 = wid * PPW

    cp_i = [
        pltpu.async_copy(x_hbm.at[b, pl.ds(p0, PPW)], idx_v.at[b], sem_i)
        for b in range(BATCH)
    ]
    wslice = wv.at[pl.ds(sid * PPW, PPW)]
    pltpu.async_copy(wpe_hbm.at[pl.ds(p0, PPW)], wslice, sem_p).wait()
    cp_c = [
        pltpu.async_copy(wslice, rows_v.at[pl.ds(b * PPW, PPW)], sem_c)
        for b in range(BATCH)
    ]
    cp_g = []
    for b in range(BATCH):
        cp_i[b].wait()
        cp_c[b].wait()
        cp_g.append(
            pltpu.async_copy(
                wte_hbm.at[idx_v.at[b]],
                rows_v.at[pl.ds(b * PPW, PPW)],
                sem_g,
                add=True,
            )
        )
    cp_s = []
    for b in range(BATCH):
        cp_g[b].wait()
        cp_s.append(
            pltpu.async_copy(
                rows_v.at[pl.ds(b * PPW, PPW)],
                out_hbm.at[b, pl.ds(p0, PPW)],
                sem_s,
            )
        )
    for cp in cp_s:
        cp.wait()


@jax.jit
def _embed(x, wte, wpe):
    mesh = plsc.VectorSubcoreMesh(core_axis_name="c", subcore_axis_name="s")
    run = functools.partial(
        pl.kernel,
        out_type=jax.ShapeDtypeStruct((BATCH, N_CTX, N_EMBED), jnp.float32),
        mesh=mesh,
        scratch_types=[
            pltpu.VMEM((BATCH, PPW), jnp.int32),
            pltpu.MemorySpace.VMEM_SHARED((NS * PPW, N_EMBED), jnp.float32),
            pltpu.VMEM((BATCH * PPW, N_EMBED), jnp.float32),
            pltpu.SemaphoreType.DMA,
            pltpu.SemaphoreType.DMA,
            pltpu.SemaphoreType.DMA,
            pltpu.SemaphoreType.DMA,
            pltpu.SemaphoreType.DMA,
        ],
    )(_sc_embed)
    return run(x, wte, wpe)


def kernel(x, wte, wpe):
    return _embed(x.astype(jnp.int32), wte, wpe)


# speedup vs baseline: 1.0428x; 1.0056x over previous
import functools

import jax
import jax.numpy as jnp
from jax import lax
from jax.experimental import pallas as pl
from jax.experimental.pallas import tpu as pltpu
from jax.experimental.pallas import tpu_sc as plsc

N_VOCAB = 100000
N_CTX = 2048
N_EMBED = 128
BATCH = 4

NC = 2
NS = 16
NW = NC * NS
PPW = N_CTX // NW


def _sc_embed(x_hbm, wte_hbm, wpe_hbm, out_hbm, idx_v, wv, rows_v, sem_i, sem_p, sem_c, sem_g, sem_s):
    wid = lax.axis_index("s") * NC + lax.axis_index("c")
    sid = lax.axis_index("s")
    p0 = wid * PPW

    cp_i = [
        pltpu.async_copy(x_hbm.at[b, pl.ds(p0, PPW)], idx_v.at[b], sem_i)
        for b in range(BATCH)
    ]
    wslice = wv.at[pl.ds(sid * PPW, PPW)]
    cp_c = [
        pltpu.async_copy(
            wpe_hbm.at[pl.ds(p0, PPW)], rows_v.at[pl.ds(0, PPW)], sem_c
        )
    ]
    pltpu.async_copy(wpe_hbm.at[pl.ds(p0, PPW)], wslice, sem_p).wait()
    cp_c += [
        pltpu.async_copy(wslice, rows_v.at[pl.ds(b * PPW, PPW)], sem_c)
        for b in range(1, BATCH)
    ]
    cp_g = []
    for b in range(BATCH):
        cp_i[b].wait()
        cp_c[b].wait()
        cp_g.append(
            pltpu.async_copy(
                wte_hbm.at[idx_v.at[b]],
                rows_v.at[pl.ds(b * PPW, PPW)],
                sem_g,
                add=True,
            )
        )
    cp_s = []
    for b in range(BATCH):
        cp_g[b].wait()
        cp_s.append(
            pltpu.async_copy(
                rows_v.at[pl.ds(b * PPW, PPW)],
                out_hbm.at[b, pl.ds(p0, PPW)],
                sem_s,
            )
        )
    for cp in cp_s:
        cp.wait()


@jax.jit
def _embed(x, wte, wpe):
    mesh = plsc.VectorSubcoreMesh(core_axis_name="c", subcore_axis_name="s")
    run = functools.partial(
        pl.kernel,
        out_type=jax.ShapeDtypeStruct((BATCH, N_CTX, N_EMBED), jnp.float32),
        mesh=mesh,
        scratch_types=[
            pltpu.VMEM((BATCH, PPW), jnp.int32),
            pltpu.MemorySpace.VMEM_SHARED((NS * PPW, N_EMBED), jnp.float32),
            pltpu.VMEM((BATCH * PPW, N_EMBED), jnp.float32),
            pltpu.SemaphoreType.DMA,
            pltpu.SemaphoreType.DMA,
            pltpu.SemaphoreType.DMA,
            pltpu.SemaphoreType.DMA,
            pltpu.SemaphoreType.DMA,
        ],
    )(_sc_embed)
    return run(x, wte, wpe)


def kernel(x, wte, wpe):
    return _embed(x.astype(jnp.int32), wte, wpe)


# speedup vs baseline: 1.0620x; 1.0184x over previous
import functools

import jax
import jax.numpy as jnp
from jax import lax
from jax.experimental import pallas as pl
from jax.experimental.pallas import tpu as pltpu
from jax.experimental.pallas import tpu_sc as plsc

N_VOCAB = 100000
N_CTX = 2048
N_EMBED = 128
BATCH = 4

NC = 2
NS = 16
NW = NC * NS
PPW = N_CTX // NW
LANES = 16


def _sc_embed(x_hbm, wte_hbm, wpe_hbm, out_hbm, idx_v, wv, rows_v, sem_i, sem_p, sem_g, sem_s):
    wid = lax.axis_index("s") * NC + lax.axis_index("c")
    p0 = wid * PPW

    cp_i = [
        pltpu.async_copy(x_hbm.at[b, pl.ds(p0, PPW)], idx_v.at[b], sem_i)
        for b in range(BATCH)
    ]
    cp_w = pltpu.async_copy(wpe_hbm.at[pl.ds(p0, PPW)], wv, sem_p)

    cp_g = []
    for b in range(BATCH):
        cp_i[b].wait()
        cp_g.append(
            pltpu.async_copy(
                wte_hbm.at[idx_v.at[b]],
                rows_v.at[pl.ds(b * PPW, PPW)],
                sem_g,
            )
        )
    cp_w.wait()

    def add_tile(b):
        @pl.loop(0, PPW)
        def _(r):
            for c in range(N_EMBED // LANES):
                sl = pl.ds(c * LANES, LANES)
                plsc.addupdate(rows_v.at[b * PPW + r, sl], wv[r, sl])

    cp_s = []
    for b in range(BATCH):
        cp_g[b].wait()
        add_tile(b)
        cp_s.append(
            pltpu.async_copy(
                rows_v.at[pl.ds(b * PPW, PPW)],
                out_hbm.at[b, pl.ds(p0, PPW)],
                sem_s,
            )
        )
    for cp in cp_s:
        cp.wait()


@jax.jit
def _embed(x, wte, wpe):
    mesh = plsc.VectorSubcoreMesh(core_axis_name="c", subcore_axis_name="s")
    run = functools.partial(
        pl.kernel,
        out_type=jax.ShapeDtypeStruct((BATCH, N_CTX, N_EMBED), jnp.float32),
        mesh=mesh,
        scratch_types=[
            pltpu.VMEM((BATCH, PPW), jnp.int32),
            pltpu.VMEM((PPW, N_EMBED), jnp.float32),
            pltpu.VMEM((BATCH * PPW, N_EMBED), jnp.float32),
            pltpu.SemaphoreType.DMA,
            pltpu.SemaphoreType.DMA,
            pltpu.SemaphoreType.DMA,
            pltpu.SemaphoreType.DMA,
        ],
    )(_sc_embed)
    return run(x, wte, wpe)


def kernel(x, wte, wpe):
    return _embed(x.astype(jnp.int32), wte, wpe)
